# baseline (speedup 1.0000x reference)
import jax
import jax.numpy as jnp
from jax.experimental import pallas as pl
from jax.experimental.pallas import tpu as pltpu

_EPS = 1e-4
_BM = 256
_NCORES = 2


def _ff_body(x_ref, b_ref, w_hbm, o_ref, w_vmem, sem):
    m = pl.program_id(1)

    @pl.when(m == 0)
    def _load_w():
        cp = pltpu.make_async_copy(w_hbm, w_vmem, sem)
        cp.start()
        cp.wait()

    half = _BM // 2
    for i in range(2):
        sl = slice(i * half, (i + 1) * half)
        xs = x_ref[sl, :]
        xb = xs.astype(jnp.bfloat16)
        acc = jax.lax.dot_general(
            xb, w_vmem[...], (((1,), (1,)), ((), ())),
            preferred_element_type=jnp.float32,
        )
        r = jnp.maximum(acc + b_ref[...], 0.0) + xs
        mu = jnp.mean(r, axis=-1, keepdims=True)
        d = r - mu
        v = jnp.mean(d * d, axis=-1, keepdims=True)
        o_ref[sl, :] = d / jnp.sqrt(v + _EPS)


@jax.jit
def kernel(x, w1, b1):
    n, f = x.shape
    w_t = w1.astype(jnp.bfloat16)
    b2d = b1.reshape(1, f)

    nm = n // (_BM * _NCORES)
    grid = (_NCORES, nm)
    return pl.pallas_call(
        _ff_body,
        grid=grid,
        in_specs=[
            pl.BlockSpec((_BM, f), lambda c, m: (c * nm + m, 0)),
            pl.BlockSpec((1, f), lambda c, m: (0, 0)),
            pl.BlockSpec(memory_space=pl.ANY),
        ],
        out_specs=pl.BlockSpec((_BM, f), lambda c, m: (c * nm + m, 0)),
        out_shape=jax.ShapeDtypeStruct((n, f), jnp.float32),
        scratch_shapes=[
            pltpu.VMEM((f, f), jnp.bfloat16),
            pltpu.SemaphoreType.DMA,
        ],
        compiler_params=pltpu.CompilerParams(
            dimension_semantics=("parallel", "arbitrary"),
            vmem_limit_bytes=60 * 1024 * 1024,
        ),
    )(x, b2d, w_t)


# speedup vs baseline: 2.3308x; 2.3308x over previous
import jax
import jax.numpy as jnp
from jax.experimental import pallas as pl
from jax.experimental.pallas import tpu as pltpu

_EPS = 1e-4
_BM = 256


def _ff_body(x_ref, b_ref, w_hbm, o_ref, w_vmem, sem):
    @pl.when(pl.program_id(0) == 0)
    def _load_w():
        cp = pltpu.make_async_copy(w_hbm, w_vmem, sem)
        cp.start()
        cp.wait()

    xb = x_ref[...].astype(jnp.bfloat16)
    acc = jax.lax.dot_general(
        xb, w_vmem[...], (((1,), (1,)), ((), ())),
        preferred_element_type=jnp.float32,
    )
    r = jnp.maximum(acc + b_ref[...], 0.0) + x_ref[...]
    mu = jnp.mean(r, axis=-1, keepdims=True)
    d = r - mu
    v = jnp.mean(d * d, axis=-1, keepdims=True)
    o_ref[...] = d / jnp.sqrt(v + _EPS)


@jax.jit
def kernel(x, w1, b1):
    n, f = x.shape
    w_bf = w1.astype(jnp.bfloat16)
    b2d = b1.reshape(1, f)

    grid = (n // _BM,)
    return pl.pallas_call(
        _ff_body,
        grid=grid,
        in_specs=[
            pl.BlockSpec((_BM, f), lambda m: (m, 0)),
            pl.BlockSpec((1, f), lambda m: (0, 0)),
            pl.BlockSpec(memory_space=pl.ANY),
        ],
        out_specs=pl.BlockSpec((_BM, f), lambda m: (m, 0)),
        out_shape=jax.ShapeDtypeStruct((n, f), jnp.float32),
        scratch_shapes=[
            pltpu.VMEM((f, f), jnp.bfloat16),
            pltpu.SemaphoreType.DMA,
        ],
        compiler_params=pltpu.CompilerParams(
            dimension_semantics=("arbitrary",),
            vmem_limit_bytes=63 * 1024 * 1024,
        ),
    )(x, b2d, w_bf)
